# baseline (speedup 1.0000x reference)
import functools

import jax
import jax.numpy as jnp
from jax import lax
from jax.experimental import pallas as pl
from jax.experimental.pallas import tpu as pltpu
from jax.experimental.pallas import tpu_sc as plsc

N_NODES = 10000
N_EDGES = 320000
N_GRAPHS = 16

NPAD = 10112
EPAD = 327680
CHUNK = 128
DEG_CHUNKS = EPAD // 32 // CHUNK
MP_CHUNKS = EPAD // 16 // CHUNK
ROWS_PER_SUB = NPAD // 16
NBUF = 4

_mesh = plsc.VectorSubcoreMesh(core_axis_name="c", subcore_axis_name="s")


def _sc_deg_body(dst_hbm, zeros_hbm, ones_hbm, out_hbm, dst_v, ones_v, sem,
                 shared):
    cid = lax.axis_index("c")
    sid = lax.axis_index("s")
    wid = cid * 16 + sid

    pltpu.sync_copy(zeros_hbm.at[pl.ds(sid * ROWS_PER_SUB, ROWS_PER_SUB)],
                    shared.at[pl.ds(sid * ROWS_PER_SUB, ROWS_PER_SUB)])
    pltpu.sync_copy(dst_hbm.at[wid], dst_v)
    pltpu.sync_copy(ones_hbm, ones_v)
    plsc.subcore_barrier()

    def step(c, carry):
        pltpu.sync_copy(ones_v, shared.at[dst_v.at[c]], add=True)
        return carry

    lax.fori_loop(0, DEG_CHUNKS, step, 0)
    plsc.subcore_barrier()
    pltpu.sync_copy(shared.at[pl.ds(sid * ROWS_PER_SUB, ROWS_PER_SUB)],
                    out_hbm.at[cid, pl.ds(sid * ROWS_PER_SUB, ROWS_PER_SUB)])


_sc_deg = pl.kernel(
    _sc_deg_body,
    out_type=jax.ShapeDtypeStruct((2, NPAD, 16), jnp.float32),
    mesh=_mesh,
    scratch_types=[
        pltpu.VMEM((DEG_CHUNKS, CHUNK), jnp.int32),
        pltpu.VMEM((CHUNK, 16), jnp.float32),
        pltpu.SemaphoreType.DMA,
        pltpu.VMEM_SHARED((NPAD, 16), jnp.float32),
    ],
    compiler_params=pltpu.CompilerParams(use_tc_tiling_on_sc=False),
)


M = 8
L = 4


M = 8
L = 4
N_PHASES = 2
PHASE_CHUNKS = MP_CHUNKS // N_PHASES


def _sc_mp_body(wh, hs_hbm, src_hbm, dst_hbm, zeros_hbm, out_hbm,
                src_v, dst_v, rows_v, gsem, ssem, shared):
    cid = lax.axis_index("c")
    sid = lax.axis_index("s")
    table = hs_hbm.at[cid]

    pltpu.sync_copy(zeros_hbm.at[pl.ds(sid * ROWS_PER_SUB, ROWS_PER_SUB)],
                    shared.at[pl.ds(sid * ROWS_PER_SUB, ROWS_PER_SUB)])
    plsc.subcore_barrier()

    def fire_g(c, b):
        pltpu.async_copy(table.at[src_v.at[c]], rows_v.at[b], gsem.at[b])

    def wait_g(c, b):
        pltpu.make_async_copy(table.at[src_v.at[c]], rows_v.at[b],
                              gsem.at[b]).wait()

    def fire_s(c, b):
        pltpu.async_copy(rows_v.at[b], shared.at[dst_v.at[c]], ssem.at[b],
                         add=True)

    def wait_s(c, b):
        pltpu.make_async_copy(rows_v.at[b], shared.at[dst_v.at[c]],
                              ssem.at[b]).wait()

    def visit(c, b, do_drain, do_fire):
        wait_g(c, b)
        fire_s(c, b)
        cp = c + L
        if do_drain:
            wait_s(cp - M, cp % M)
        if do_fire:
            fire_g(cp, cp % M)

    for p in range(N_PHASES):
        pltpu.sync_copy(src_hbm.at[sid, pl.ds(p * PHASE_CHUNKS, PHASE_CHUNKS)],
                        src_v)
        pltpu.sync_copy(dst_hbm.at[sid, pl.ds(p * PHASE_CHUNKS, PHASE_CHUNKS)],
                        dst_v)

        for b in range(L):
            fire_g(b, b)
        for b in range(M):
            visit(b, b, do_drain=(b >= L), do_fire=True)

        def group(g, carry):
            for b in range(M):
                visit(g * M + b, b, do_drain=True, do_fire=True)
            return carry

        lax.fori_loop(1, PHASE_CHUNKS // M - 1, group, 0)
        for b in range(M):
            c = PHASE_CHUNKS - M + b
            visit(c, b, do_drain=True, do_fire=(b < M - L))
        for b in range(L):
            c = PHASE_CHUNKS - L + b
            wait_s(c, c % M)

    plsc.subcore_barrier()
    pltpu.sync_copy(shared.at[pl.ds(sid * ROWS_PER_SUB, ROWS_PER_SUB)],
                    out_hbm.at[cid, pl.ds(sid * ROWS_PER_SUB, ROWS_PER_SUB)])


def _make_sc_mp(wh):
    return pl.kernel(
        functools.partial(_sc_mp_body, wh),
        out_type=jax.ShapeDtypeStruct((2, NPAD, wh), jnp.float32),
        mesh=_mesh,
        scratch_types=[
            pltpu.VMEM((PHASE_CHUNKS, CHUNK), jnp.int32),
            pltpu.VMEM((PHASE_CHUNKS, CHUNK), jnp.int32),
            pltpu.VMEM((M, CHUNK, wh), jnp.float32),
            pltpu.SemaphoreType.DMA((M,)),
            pltpu.SemaphoreType.DMA((M,)),
            pltpu.VMEM_SHARED((NPAD, wh), jnp.float32),
        ],
        compiler_params=pltpu.CompilerParams(use_tc_tiling_on_sc=False),
    )


_sc_mp32 = _make_sc_mp(32)
_sc_mp64 = _make_sc_mp(64)


def _mm1_body(x_ref, w1_ref, degp_ref, hs1_ref, dinv_ref):
    degp = degp_ref[...]
    deg = 1.0 + degp[0, :, 0:1] + degp[1, :, 0:1]
    dinv = lax.rsqrt(deg)
    h = jnp.dot(x_ref[...], w1_ref[...], preferred_element_type=jnp.float32)
    hs = h * dinv
    hs1_ref[0] = hs[:, :32]
    hs1_ref[1] = hs[:, 32:]
    dinv_ref[...] = dinv


def _mm2_body(p1_ref, hs1_ref, dinv_ref, b1_ref, w2_ref, hs2_ref):
    p1 = p1_ref[...]
    hs1 = hs1_ref[...]
    dinv = dinv_ref[...]
    acc = jnp.concatenate([p1[0] + hs1[0], p1[1] + hs1[1]], axis=1)
    z = jax.nn.relu(dinv * acc + b1_ref[...])
    hs2 = jnp.dot(z, w2_ref[...], preferred_element_type=jnp.float32) * dinv
    hs2_ref[0] = hs2[:, :64]
    hs2_ref[1] = hs2[:, 64:]


def _pool_body(p2_ref, hs2_ref, dinv_ref, b2_ref, batch_ref, out_ref):
    p2 = p2_ref[...]
    hs2 = hs2_ref[...]
    h2 = dinv_ref[...] * jnp.concatenate(
        [p2[0] + hs2[0], p2[1] + hs2[1]], axis=1)
    gids = lax.broadcasted_iota(jnp.int32, (N_GRAPHS, NPAD), 0)
    onehot = (gids == batch_ref[...]).astype(jnp.float32)
    pooled = jnp.dot(onehot, h2, preferred_element_type=jnp.float32)
    counts = jnp.maximum(jnp.sum(onehot, axis=1, keepdims=True), 1.0)
    out_ref[...] = pooled / counts + b2_ref[...]


_tc_mm1 = pl.pallas_call(
    _mm1_body,
    out_shape=(jax.ShapeDtypeStruct((2, NPAD, 32), jnp.float32),
               jax.ShapeDtypeStruct((NPAD, 1), jnp.float32)),
)

_tc_mm2 = pl.pallas_call(
    _mm2_body,
    out_shape=jax.ShapeDtypeStruct((2, NPAD, 64), jnp.float32),
)

_tc_pool = pl.pallas_call(
    _pool_body,
    out_shape=jax.ShapeDtypeStruct((N_GRAPHS, 128), jnp.float32),
)


@jax.jit
def kernel(x, edge_index, batch, W1, b1, W2, b2):
    ei = edge_index.astype(jnp.int32)
    pad = jnp.full((EPAD - N_EDGES,), N_NODES, jnp.int32)
    srcp = jnp.concatenate([ei[0], pad])
    dstp = jnp.concatenate([ei[1], pad])
    src_deg = dstp.reshape(32, DEG_CHUNKS, CHUNK)
    src2 = srcp.reshape(16, MP_CHUNKS, CHUNK)
    dst2 = dstp.reshape(16, MP_CHUNKS, CHUNK)
    xp = jnp.pad(x.astype(jnp.float32), ((0, NPAD - N_NODES), (0, 0)))
    batchp = jnp.pad(batch.astype(jnp.int32), (0, NPAD - N_NODES),
                     constant_values=-1).reshape(1, NPAD)

    zeros16 = jnp.zeros((NPAD, 16), jnp.float32)
    zeros32 = jnp.zeros((NPAD, 32), jnp.float32)
    zeros64 = jnp.zeros((NPAD, 64), jnp.float32)
    ones16 = jnp.ones((CHUNK, 16), jnp.float32)

    degp = _sc_deg(src_deg, zeros16, ones16)
    hs1, dinv = _tc_mm1(xp, W1, degp)
    p1 = _sc_mp32(hs1, src2, dst2, zeros32)
    hs2 = _tc_mm2(p1, hs1, dinv, b1.reshape(1, 64), W2)
    p2 = _sc_mp64(hs2, src2, dst2, zeros64)
    return _tc_pool(p2, hs2, dinv, b2.reshape(1, 128), batchp)


# speedup vs baseline: 1.5094x; 1.5094x over previous
import functools

import jax
import jax.numpy as jnp
from jax import lax
from jax.experimental import pallas as pl
from jax.experimental.pallas import tpu as pltpu
from jax.experimental.pallas import tpu_sc as plsc

N_NODES = 10000
N_EDGES = 320000
N_GRAPHS = 16

NPAD = 10112
EPAD = 327680
CHUNK = 128
DEG_CHUNKS = EPAD // 32 // CHUNK
MP_CHUNKS = EPAD // 16 // CHUNK
ROWS_PER_SUB = NPAD // 16
NBUF = 4

_mesh = plsc.VectorSubcoreMesh(core_axis_name="c", subcore_axis_name="s")


def _sc_deg_body(dst_hbm, zeros_hbm, ones_hbm, out_hbm, dst_v, ones_v, sem,
                 shared):
    cid = lax.axis_index("c")
    sid = lax.axis_index("s")
    wid = cid * 16 + sid

    pltpu.sync_copy(zeros_hbm.at[pl.ds(sid * ROWS_PER_SUB, ROWS_PER_SUB)],
                    shared.at[pl.ds(sid * ROWS_PER_SUB, ROWS_PER_SUB)])
    pltpu.sync_copy(dst_hbm.at[wid], dst_v)
    pltpu.sync_copy(ones_hbm, ones_v)
    plsc.subcore_barrier()

    def step(c, carry):
        pltpu.sync_copy(ones_v, shared.at[dst_v.at[c]], add=True)
        return carry

    lax.fori_loop(0, DEG_CHUNKS, step, 0)
    plsc.subcore_barrier()
    pltpu.sync_copy(shared.at[pl.ds(sid * ROWS_PER_SUB, ROWS_PER_SUB)],
                    out_hbm.at[cid, pl.ds(sid * ROWS_PER_SUB, ROWS_PER_SUB)])


_sc_deg = pl.kernel(
    _sc_deg_body,
    out_type=jax.ShapeDtypeStruct((2, NPAD, 16), jnp.float32),
    mesh=_mesh,
    scratch_types=[
        pltpu.VMEM((DEG_CHUNKS, CHUNK), jnp.int32),
        pltpu.VMEM((CHUNK, 16), jnp.float32),
        pltpu.SemaphoreType.DMA,
        pltpu.VMEM_SHARED((NPAD, 16), jnp.float32),
    ],
    compiler_params=pltpu.CompilerParams(use_tc_tiling_on_sc=False),
)


M = 8
L = 4


M = 8
L = 4
N_PHASES = 2
PHASE_CHUNKS = MP_CHUNKS // N_PHASES


def _sc_mp_body(wh, hs_hbm, src_hbm, dst_hbm, zeros_hbm, out_hbm,
                src_v, dst_v, rows_v, gsem, ssem, shared):
    cid = lax.axis_index("c")
    sid = lax.axis_index("s")
    table = hs_hbm.at[cid]

    pltpu.sync_copy(zeros_hbm.at[pl.ds(sid * ROWS_PER_SUB, ROWS_PER_SUB)],
                    shared.at[pl.ds(sid * ROWS_PER_SUB, ROWS_PER_SUB)])
    plsc.subcore_barrier()

    def fire_g(c, b):
        pltpu.async_copy(table.at[src_v.at[c]], rows_v.at[b], gsem.at[b])

    def wait_g(c, b):
        pltpu.make_async_copy(table.at[src_v.at[c]], rows_v.at[b],
                              gsem.at[b]).wait()

    def fire_s(c, b):
        pltpu.async_copy(rows_v.at[b], shared.at[dst_v.at[c]], ssem.at[b],
                         add=True)

    def wait_s(c, b):
        pltpu.make_async_copy(rows_v.at[b], shared.at[dst_v.at[c]],
                              ssem.at[b]).wait()

    def visit(c, b, do_drain, do_fire):
        wait_g(c, b)
        fire_s(c, b)
        cp = c + L
        if do_drain:
            wait_s(cp - M, cp % M)
        if do_fire:
            fire_g(cp, cp % M)

    for p in range(N_PHASES):
        pltpu.sync_copy(src_hbm.at[sid, pl.ds(p * PHASE_CHUNKS, PHASE_CHUNKS)],
                        src_v)
        pltpu.sync_copy(dst_hbm.at[sid, pl.ds(p * PHASE_CHUNKS, PHASE_CHUNKS)],
                        dst_v)

        for b in range(L):
            fire_g(b, b)
        for b in range(M):
            visit(b, b, do_drain=(b >= L), do_fire=True)

        def group(g, carry):
            for b in range(M):
                visit(g * M + b, b, do_drain=True, do_fire=True)
            return carry

        lax.fori_loop(1, PHASE_CHUNKS // M - 1, group, 0)
        for b in range(M):
            c = PHASE_CHUNKS - M + b
            visit(c, b, do_drain=True, do_fire=(b < M - L))
        for b in range(L):
            c = PHASE_CHUNKS - L + b
            wait_s(c, c % M)

    plsc.subcore_barrier()
    pltpu.sync_copy(shared.at[pl.ds(sid * ROWS_PER_SUB, ROWS_PER_SUB)],
                    out_hbm.at[cid, pl.ds(sid * ROWS_PER_SUB, ROWS_PER_SUB)])


def _make_sc_mp(wh):
    return pl.kernel(
        functools.partial(_sc_mp_body, wh),
        out_type=jax.ShapeDtypeStruct((2, NPAD, wh), jnp.bfloat16),
        mesh=_mesh,
        scratch_types=[
            pltpu.VMEM((PHASE_CHUNKS, CHUNK), jnp.int32),
            pltpu.VMEM((PHASE_CHUNKS, CHUNK), jnp.int32),
            pltpu.VMEM((M, CHUNK, wh), jnp.bfloat16),
            pltpu.SemaphoreType.DMA((M,)),
            pltpu.SemaphoreType.DMA((M,)),
            pltpu.VMEM_SHARED((NPAD, wh), jnp.bfloat16),
        ],
        compiler_params=pltpu.CompilerParams(use_tc_tiling_on_sc=False),
    )


_sc_mp32 = _make_sc_mp(32)
_sc_mp64 = _make_sc_mp(64)


def _mm1_body(x_ref, w1_ref, degp_ref, hs1_ref, dinv_ref):
    degp = degp_ref[...]
    deg = 1.0 + degp[0, :, 0:1] + degp[1, :, 0:1]
    dinv = lax.rsqrt(deg)
    h = jnp.dot(x_ref[...], w1_ref[...], preferred_element_type=jnp.float32)
    hs = (h * dinv).astype(jnp.bfloat16)
    hs1_ref[0] = hs[:, :32]
    hs1_ref[1] = hs[:, 32:]
    dinv_ref[...] = dinv


def _mm2_body(p1_ref, hs1_ref, dinv_ref, b1_ref, w2_ref, hs2_ref):
    p1 = p1_ref[...].astype(jnp.float32)
    hs1 = hs1_ref[...].astype(jnp.float32)
    dinv = dinv_ref[...]
    acc = jnp.concatenate([p1[0] + hs1[0], p1[1] + hs1[1]], axis=1)
    z = jax.nn.relu(dinv * acc + b1_ref[...])
    hs2 = ((jnp.dot(z, w2_ref[...], preferred_element_type=jnp.float32))
           * dinv).astype(jnp.bfloat16)
    hs2_ref[0] = hs2[:, :64]
    hs2_ref[1] = hs2[:, 64:]


def _pool_body(p2_ref, hs2_ref, dinv_ref, b2_ref, batch_ref, out_ref):
    p2 = p2_ref[...].astype(jnp.float32)
    hs2 = hs2_ref[...].astype(jnp.float32)
    h2 = dinv_ref[...] * jnp.concatenate(
        [p2[0] + hs2[0], p2[1] + hs2[1]], axis=1)
    gids = lax.broadcasted_iota(jnp.int32, (N_GRAPHS, NPAD), 0)
    onehot = (gids == batch_ref[...]).astype(jnp.float32)
    pooled = jnp.dot(onehot, h2, preferred_element_type=jnp.float32)
    counts = jnp.maximum(jnp.sum(onehot, axis=1, keepdims=True), 1.0)
    out_ref[...] = pooled / counts + b2_ref[...]


_tc_mm1 = pl.pallas_call(
    _mm1_body,
    out_shape=(jax.ShapeDtypeStruct((2, NPAD, 32), jnp.bfloat16),
               jax.ShapeDtypeStruct((NPAD, 1), jnp.float32)),
)

_tc_mm2 = pl.pallas_call(
    _mm2_body,
    out_shape=jax.ShapeDtypeStruct((2, NPAD, 64), jnp.bfloat16),
)

_tc_pool = pl.pallas_call(
    _pool_body,
    out_shape=jax.ShapeDtypeStruct((N_GRAPHS, 128), jnp.float32),
)


@jax.jit
def kernel(x, edge_index, batch, W1, b1, W2, b2):
    ei = edge_index.astype(jnp.int32)
    pad = jnp.full((EPAD - N_EDGES,), N_NODES, jnp.int32)
    srcp = jnp.concatenate([ei[0], pad])
    dstp = jnp.concatenate([ei[1], pad])
    src_deg = dstp.reshape(32, DEG_CHUNKS, CHUNK)
    src2 = srcp.reshape(16, MP_CHUNKS, CHUNK)
    dst2 = dstp.reshape(16, MP_CHUNKS, CHUNK)
    xp = jnp.pad(x.astype(jnp.float32), ((0, NPAD - N_NODES), (0, 0)))
    batchp = jnp.pad(batch.astype(jnp.int32), (0, NPAD - N_NODES),
                     constant_values=-1).reshape(1, NPAD)

    zeros16 = jnp.zeros((NPAD, 16), jnp.float32)
    zeros32 = jnp.zeros((NPAD, 32), jnp.bfloat16)
    zeros64 = jnp.zeros((NPAD, 64), jnp.bfloat16)
    ones16 = jnp.ones((CHUNK, 16), jnp.float32)

    degp = _sc_deg(src_deg, zeros16, ones16)
    hs1, dinv = _tc_mm1(xp, W1, degp)
    p1 = _sc_mp32(hs1, src2, dst2, zeros32)
    hs2 = _tc_mm2(p1, hs1, dinv, b1.reshape(1, 64), W2)
    p2 = _sc_mp64(hs2, src2, dst2, zeros64)
    return _tc_pool(p2, hs2, dinv, b2.reshape(1, 128), batchp)
